# baseline (speedup 1.0000x reference)
import functools

import jax
import jax.numpy as jnp
from jax import lax
from jax.experimental import pallas as pl
from jax.experimental.pallas import tpu as pltpu
from jax.experimental.pallas import tpu_sc as plsc

H = 32
G = 64
NCORE = 2
NSUB = 16
NW = NCORE * NSUB
CHUNK = 128
KB = 4
BM = 2000


def _edge_pass(y, src2d, dst2d, attr2d, zeros, n_nodes, cpw):
    npt = n_nodes // NSUB
    nblk = cpw // KB
    mesh = plsc.VectorSubcoreMesh(core_axis_name="c", subcore_axis_name="s")

    @functools.partial(
        pl.kernel,
        mesh=mesh,
        compiler_params=pltpu.CompilerParams(use_tc_tiling_on_sc=False),
        out_type=jax.ShapeDtypeStruct((NCORE, n_nodes, H), jnp.float32),
        scratch_types=[
            pltpu.VMEM((cpw, CHUNK), jnp.int32),
            pltpu.VMEM((cpw, CHUNK), jnp.int32),
            pltpu.VMEM((cpw, CHUNK), jnp.float32),
            pltpu.VMEM((KB, CHUNK, H), jnp.float32),
            pltpu.VMEM_SHARED((n_nodes, H), jnp.float32),
            pltpu.VMEM_SHARED((n_nodes, H), jnp.float32),
            pltpu.SemaphoreType.DMA,
            pltpu.SemaphoreType.DMA,
        ],
    )
    def body(y_hbm, src_hbm, dst_hbm, attr_hbm, z_hbm, out_hbm,
             srcb, dstb, attrb, rows, agg, ysh, sem, sem2):
        cid = lax.axis_index("c")
        sid = lax.axis_index("s")
        wid = sid * NCORE + cid
        pltpu.sync_copy(z_hbm.at[pl.ds(sid * npt, npt)],
                        agg.at[pl.ds(sid * npt, npt)])
        pltpu.sync_copy(y_hbm.at[pl.ds(sid * npt, npt)],
                        ysh.at[pl.ds(sid * npt, npt)])
        wrow = wid * cpw
        pltpu.sync_copy(src_hbm.at[pl.ds(wrow, cpw)], srcb)
        pltpu.sync_copy(dst_hbm.at[pl.ds(wrow, cpw)], dstb)
        pltpu.sync_copy(attr_hbm.at[pl.ds(wrow, cpw)], attrb)
        plsc.subcore_barrier()

        def blk(b, carry):
            c0 = b * KB
            gh = [pltpu.async_copy(ysh.at[srcb.at[c0 + j]], rows.at[j], sem)
                  for j in range(KB)]
            sh = []
            for j in range(KB):
                gh[j].wait()

                def grp(g, _, j=j):
                    e0 = g * 16
                    av = attrb[c0 + j, pl.ds(e0, 16)]
                    for i in range(16):
                        s = jnp.take_along_axis(
                            av, jnp.full((16,), i, jnp.int32), axis=0)
                        rows[j, e0 + i, pl.ds(0, 16)] = (
                            rows[j, e0 + i, pl.ds(0, 16)] * s)
                        rows[j, e0 + i, pl.ds(16, 16)] = (
                            rows[j, e0 + i, pl.ds(16, 16)] * s)
                    return 0

                lax.fori_loop(0, CHUNK // 16, grp, 0, unroll=CHUNK // 16)
                sh.append(pltpu.async_copy(
                    rows.at[j], agg.at[dstb.at[c0 + j]], sem2, add=True))
            for h in sh:
                h.wait()
            return carry

        lax.fori_loop(0, nblk, blk, 0)
        plsc.subcore_barrier()
        pltpu.sync_copy(agg.at[pl.ds(sid * npt, npt)],
                        out_hbm.at[cid, pl.ds(sid * npt, npt)])

    return body(y, src2d, dst2d, attr2d, zeros)


def _proj_body(x_ref, w_ref, y_ref, r_ref):
    acc = jnp.dot(x_ref[...], w_ref[...], preferred_element_type=jnp.float32)
    y_ref[...] = acc[:, :H]
    r_ref[...] = acc[:, H:]


def _project(x, wcat):
    n, f = x.shape
    return pl.pallas_call(
        _proj_body,
        grid=(n // BM,),
        in_specs=[pl.BlockSpec((BM, f), lambda i: (i, 0)),
                  pl.BlockSpec((f, 2 * H), lambda i: (0, 0))],
        out_specs=[pl.BlockSpec((BM, H), lambda i: (i, 0)),
                   pl.BlockSpec((BM, H), lambda i: (i, 0))],
        out_shape=[jax.ShapeDtypeStruct((n, H), jnp.float32)] * 2,
    )(x, wcat)


def _combine_body(aggp_ref, r_ref, b_ref, w_ref, y_ref, r2_ref):
    h = jnp.maximum(aggp_ref[0] + aggp_ref[1] + r_ref[...] + b_ref[...], 0.0)
    acc = jnp.dot(h, w_ref[...], preferred_element_type=jnp.float32)
    y_ref[...] = acc[:, :H]
    r2_ref[...] = acc[:, H:]


def _combine_project(aggp, r1, b1, wcat):
    n = r1.shape[0]
    return pl.pallas_call(
        _combine_body,
        grid=(n // BM,),
        in_specs=[pl.BlockSpec((NCORE, BM, H), lambda i: (0, i, 0)),
                  pl.BlockSpec((BM, H), lambda i: (i, 0)),
                  pl.BlockSpec((1, H), lambda i: (0, 0)),
                  pl.BlockSpec((H, 2 * H), lambda i: (0, 0))],
        out_specs=[pl.BlockSpec((BM, H), lambda i: (i, 0)),
                   pl.BlockSpec((BM, H), lambda i: (i, 0))],
        out_shape=[jax.ShapeDtypeStruct((n, H), jnp.float32)] * 2,
    )(aggp, r1, b1, wcat)


def _final_body(aggp_ref, r_ref, b_ref, batch_ref, wl1_ref, bl1_ref,
                wl2_ref, bl2_ref, wo_ref, bo_ref, out_ref, sums_ref, cnt_ref):
    i = pl.program_id(0)
    nb = pl.num_programs(0)

    @pl.when(i == 0)
    def _init():
        sums_ref[...] = jnp.zeros_like(sums_ref)
        cnt_ref[...] = jnp.zeros_like(cnt_ref)

    h = jnp.maximum(aggp_ref[0] + aggp_ref[1] + r_ref[...] + b_ref[...], 0.0)
    bvec = batch_ref[0, 0, :]
    oh = (lax.broadcasted_iota(jnp.int32, (G, BM), 0) == bvec[None, :]
          ).astype(jnp.float32)
    sums_ref[...] += jnp.dot(oh, h, preferred_element_type=jnp.float32)
    cnt_ref[...] += jnp.broadcast_to(
        jnp.sum(oh, axis=1, keepdims=True), cnt_ref.shape)

    @pl.when(i == nb - 1)
    def _fin():
        pooled = sums_ref[...] / jnp.maximum(cnt_ref[:, :1], 1.0)
        g1 = jnp.maximum(
            jnp.dot(pooled, wl1_ref[...], preferred_element_type=jnp.float32)
            + bl1_ref[...], 0.0)
        g2 = jnp.maximum(
            jnp.dot(g1, wl2_ref[...], preferred_element_type=jnp.float32)
            + bl2_ref[...], 0.0)
        out_ref[...] = (jnp.dot(g2, wo_ref[...],
                                preferred_element_type=jnp.float32)
                        + bo_ref[...])


def _final(aggp, r2, b2, batch3d, wl1, bl1, wl2, bl2, wo, bo):
    n = r2.shape[0]
    return pl.pallas_call(
        _final_body,
        grid=(n // BM,),
        in_specs=[pl.BlockSpec((NCORE, BM, H), lambda i: (0, i, 0)),
                  pl.BlockSpec((BM, H), lambda i: (i, 0)),
                  pl.BlockSpec((1, H), lambda i: (0, 0)),
                  pl.BlockSpec((1, 1, BM), lambda i: (i, 0, 0)),
                  pl.BlockSpec((H, H), lambda i: (0, 0)),
                  pl.BlockSpec((1, H), lambda i: (0, 0)),
                  pl.BlockSpec((H, 16), lambda i: (0, 0)),
                  pl.BlockSpec((1, 16), lambda i: (0, 0)),
                  pl.BlockSpec((16, 1), lambda i: (0, 0)),
                  pl.BlockSpec((1, 1), lambda i: (0, 0))],
        out_specs=pl.BlockSpec((G, 1), lambda i: (0, 0)),
        out_shape=jax.ShapeDtypeStruct((G, 1), jnp.float32),
        scratch_shapes=[pltpu.VMEM((G, H), jnp.float32),
                        pltpu.VMEM((G, 128), jnp.float32)],
    )(aggp, r2, b2, batch3d, wl1, bl1, wl2, bl2, wo, bo)


def kernel(x, edge_index, edge_attr, batch,
           W_rel1, b_rel1, W_root1, W_rel2, b_rel2, W_root2,
           W_l1, b_l1, W_l2, b_l2, W_out, b_out):
    n, _ = x.shape
    e = edge_index.shape[1]

    per = NW * CHUNK * KB
    ep = ((e + per - 1) // per) * per
    cpw = ep // (NW * CHUNK)
    pad = ep - e
    src = jnp.concatenate(
        [edge_index[0], jnp.zeros((pad,), jnp.int32)]).reshape(ep // CHUNK, CHUNK)
    dst = jnp.concatenate(
        [edge_index[1], jnp.zeros((pad,), jnp.int32)]).reshape(ep // CHUNK, CHUNK)
    attr = jnp.concatenate(
        [edge_attr, jnp.zeros((pad,), jnp.float32)]).reshape(ep // CHUNK, CHUNK)
    zeros = jnp.zeros((n, H), jnp.float32)

    wcat1 = jnp.concatenate([W_rel1, W_root1], axis=1)
    wcat2 = jnp.concatenate([W_rel2, W_root2], axis=1)

    y1, r1 = _project(x, wcat1)
    aggp1 = _edge_pass(y1, src, dst, attr, zeros, n, cpw)
    y2, r2 = _combine_project(aggp1, r1, b_rel1.reshape(1, H), wcat2)
    aggp2 = _edge_pass(y2, src, dst, attr, zeros, n, cpw)
    return _final(aggp2, r2, b_rel2.reshape(1, H),
                  batch.reshape(n // BM, 1, BM),
                  W_l1, b_l1.reshape(1, H), W_l2, b_l2.reshape(1, 16),
                  W_out, b_out.reshape(1, 1))


# speedup vs baseline: 1.1448x; 1.1448x over previous
import functools

import jax
import jax.numpy as jnp
from jax import lax
from jax.experimental import pallas as pl
from jax.experimental.pallas import tpu as pltpu
from jax.experimental.pallas import tpu_sc as plsc

H = 32
G = 64
NCORE = 2
NSUB = 16
NW = NCORE * NSUB
CHUNK = 80
KB = 5
BM = 2000


def _edge_pass(y, ei3, attr2d, zeros, n_nodes, cpw):
    npt = n_nodes // NSUB
    nblk = cpw // KB
    mesh = plsc.VectorSubcoreMesh(core_axis_name="c", subcore_axis_name="s")

    @functools.partial(
        pl.kernel,
        mesh=mesh,
        compiler_params=pltpu.CompilerParams(use_tc_tiling_on_sc=False),
        out_type=jax.ShapeDtypeStruct((NCORE, n_nodes, H), jnp.float32),
        scratch_types=[
            pltpu.VMEM((cpw, CHUNK), jnp.int32),
            pltpu.VMEM((cpw, CHUNK), jnp.int32),
            pltpu.VMEM((cpw, CHUNK), jnp.float32),
            pltpu.VMEM((KB, CHUNK, H), jnp.float32),
            pltpu.VMEM_SHARED((n_nodes, H), jnp.float32),
            pltpu.VMEM_SHARED((n_nodes, H), jnp.float32),
            pltpu.SemaphoreType.DMA,
            pltpu.SemaphoreType.DMA,
        ],
    )
    def body(y_hbm, ei_hbm, attr_hbm, z_hbm, out_hbm,
             srcb, dstb, attrb, rows, agg, ysh, sem, sem2):
        cid = lax.axis_index("c")
        sid = lax.axis_index("s")
        wid = sid * NCORE + cid
        pltpu.sync_copy(z_hbm.at[pl.ds(sid * npt, npt)],
                        agg.at[pl.ds(sid * npt, npt)])
        pltpu.sync_copy(y_hbm.at[pl.ds(sid * npt, npt)],
                        ysh.at[pl.ds(sid * npt, npt)])
        wrow = wid * cpw
        pltpu.sync_copy(ei_hbm.at[0, pl.ds(wrow, cpw)], srcb)
        pltpu.sync_copy(ei_hbm.at[1, pl.ds(wrow, cpw)], dstb)
        pltpu.sync_copy(attr_hbm.at[pl.ds(wrow, cpw)], attrb)
        plsc.subcore_barrier()

        def blk(b, carry):
            c0 = b * KB
            gh = [pltpu.async_copy(ysh.at[srcb.at[c0 + j]], rows.at[j], sem)
                  for j in range(KB)]
            sh = []
            for j in range(KB):
                gh[j].wait()

                def grp(g, _, j=j):
                    e0 = g * 16
                    av = attrb[c0 + j, pl.ds(e0, 16)]
                    for i in range(16):
                        s = jnp.take_along_axis(
                            av, jnp.full((16,), i, jnp.int32), axis=0)
                        rows[j, e0 + i, pl.ds(0, 16)] = (
                            rows[j, e0 + i, pl.ds(0, 16)] * s)
                        rows[j, e0 + i, pl.ds(16, 16)] = (
                            rows[j, e0 + i, pl.ds(16, 16)] * s)
                    return 0

                lax.fori_loop(0, CHUNK // 16, grp, 0)
                sh.append(pltpu.async_copy(
                    rows.at[j], agg.at[dstb.at[c0 + j]], sem2, add=True))
            for h in sh:
                h.wait()
            return carry

        lax.fori_loop(0, nblk, blk, 0)
        plsc.subcore_barrier()
        pltpu.sync_copy(agg.at[pl.ds(sid * npt, npt)],
                        out_hbm.at[cid, pl.ds(sid * npt, npt)])

    return body(y, ei3, attr2d, zeros)


def _proj_body(x_ref, w_ref, y_ref, r_ref):
    acc = jnp.dot(x_ref[...], w_ref[...], preferred_element_type=jnp.float32)
    y_ref[...] = acc[:, :H]
    r_ref[...] = acc[:, H:]


def _project(x, wcat):
    n, f = x.shape
    return pl.pallas_call(
        _proj_body,
        grid=(n // BM,),
        in_specs=[pl.BlockSpec((BM, f), lambda i: (i, 0)),
                  pl.BlockSpec((f, 2 * H), lambda i: (0, 0))],
        out_specs=[pl.BlockSpec((BM, H), lambda i: (i, 0)),
                   pl.BlockSpec((BM, H), lambda i: (i, 0))],
        out_shape=[jax.ShapeDtypeStruct((n, H), jnp.float32)] * 2,
    )(x, wcat)


def _combine_body(aggp_ref, r_ref, b_ref, w_ref, y_ref, r2_ref):
    h = jnp.maximum(aggp_ref[0] + aggp_ref[1] + r_ref[...] + b_ref[...], 0.0)
    acc = jnp.dot(h, w_ref[...], preferred_element_type=jnp.float32)
    y_ref[...] = acc[:, :H]
    r2_ref[...] = acc[:, H:]


def _combine_project(aggp, r1, b1, wcat):
    n = r1.shape[0]
    return pl.pallas_call(
        _combine_body,
        grid=(n // BM,),
        in_specs=[pl.BlockSpec((NCORE, BM, H), lambda i: (0, i, 0)),
                  pl.BlockSpec((BM, H), lambda i: (i, 0)),
                  pl.BlockSpec((1, H), lambda i: (0, 0)),
                  pl.BlockSpec((H, 2 * H), lambda i: (0, 0))],
        out_specs=[pl.BlockSpec((BM, H), lambda i: (i, 0)),
                   pl.BlockSpec((BM, H), lambda i: (i, 0))],
        out_shape=[jax.ShapeDtypeStruct((n, H), jnp.float32)] * 2,
    )(aggp, r1, b1, wcat)


def _final_body(aggp_ref, r_ref, b_ref, batch_ref, wl1_ref, bl1_ref,
                wl2_ref, bl2_ref, wo_ref, bo_ref, out_ref, sums_ref, cnt_ref):
    i = pl.program_id(0)
    nb = pl.num_programs(0)

    @pl.when(i == 0)
    def _init():
        sums_ref[...] = jnp.zeros_like(sums_ref)
        cnt_ref[...] = jnp.zeros_like(cnt_ref)

    h = jnp.maximum(aggp_ref[0] + aggp_ref[1] + r_ref[...] + b_ref[...], 0.0)
    bvec = batch_ref[0, 0, :]
    oh = (lax.broadcasted_iota(jnp.int32, (G, BM), 0) == bvec[None, :]
          ).astype(jnp.float32)
    sums_ref[...] += jnp.dot(oh, h, preferred_element_type=jnp.float32)
    cnt_ref[...] += jnp.broadcast_to(
        jnp.sum(oh, axis=1, keepdims=True), cnt_ref.shape)

    @pl.when(i == nb - 1)
    def _fin():
        pooled = sums_ref[...] / jnp.maximum(cnt_ref[:, :1], 1.0)
        g1 = jnp.maximum(
            jnp.dot(pooled, wl1_ref[...], preferred_element_type=jnp.float32)
            + bl1_ref[...], 0.0)
        g2 = jnp.maximum(
            jnp.dot(g1, wl2_ref[...], preferred_element_type=jnp.float32)
            + bl2_ref[...], 0.0)
        out_ref[...] = (jnp.dot(g2, wo_ref[...],
                                preferred_element_type=jnp.float32)
                        + bo_ref[...])


def _final(aggp, r2, b2, batch3d, wl1, bl1, wl2, bl2, wo, bo):
    n = r2.shape[0]
    return pl.pallas_call(
        _final_body,
        grid=(n // BM,),
        in_specs=[pl.BlockSpec((NCORE, BM, H), lambda i: (0, i, 0)),
                  pl.BlockSpec((BM, H), lambda i: (i, 0)),
                  pl.BlockSpec((1, H), lambda i: (0, 0)),
                  pl.BlockSpec((1, 1, BM), lambda i: (i, 0, 0)),
                  pl.BlockSpec((H, H), lambda i: (0, 0)),
                  pl.BlockSpec((1, H), lambda i: (0, 0)),
                  pl.BlockSpec((H, 16), lambda i: (0, 0)),
                  pl.BlockSpec((1, 16), lambda i: (0, 0)),
                  pl.BlockSpec((16, 1), lambda i: (0, 0)),
                  pl.BlockSpec((1, 1), lambda i: (0, 0))],
        out_specs=pl.BlockSpec((G, 1), lambda i: (0, 0)),
        out_shape=jax.ShapeDtypeStruct((G, 1), jnp.float32),
        scratch_shapes=[pltpu.VMEM((G, H), jnp.float32),
                        pltpu.VMEM((G, 128), jnp.float32)],
    )(aggp, r2, b2, batch3d, wl1, bl1, wl2, bl2, wo, bo)


def kernel(x, edge_index, edge_attr, batch,
           W_rel1, b_rel1, W_root1, W_rel2, b_rel2, W_root2,
           W_l1, b_l1, W_l2, b_l2, W_out, b_out):
    n, _ = x.shape
    e = edge_index.shape[1]

    cpw = e // (NW * CHUNK)
    ei3 = edge_index.reshape(2, e // CHUNK, CHUNK)
    attr = edge_attr.reshape(e // CHUNK, CHUNK)
    zeros = jnp.zeros((n, H), jnp.float32)

    wcat1 = jnp.concatenate([W_rel1, W_root1], axis=1)
    wcat2 = jnp.concatenate([W_rel2, W_root2], axis=1)

    y1, r1 = _project(x, wcat1)
    aggp1 = _edge_pass(y1, ei3, attr, zeros, n, cpw)
    y2, r2 = _combine_project(aggp1, r1, b_rel1.reshape(1, H), wcat2)
    aggp2 = _edge_pass(y2, ei3, attr, zeros, n, cpw)
    return _final(aggp2, r2, b_rel2.reshape(1, H),
                  batch.reshape(n // BM, 1, BM),
                  W_l1, b_l1.reshape(1, H), W_l2, b_l2.reshape(1, 16),
                  W_out, b_out.reshape(1, 1))
